# Initial kernel scaffold; baseline (speedup 1.0000x reference)
import jax
import jax.numpy as jnp
from jax.experimental import pallas as pl


def kernel(feat0, feat1, feat2, feat3, feat4, conv_w, conv_b, cls_w, cls_b, bbox_w, bbox_b, image_h, image_w):
    raise NotImplementedError("write your pallas kernel here")


# speedup vs baseline: 7.1879x; 7.1879x over previous
import functools

import numpy as np
import jax
import jax.numpy as jnp
from jax.experimental import pallas as pl

BATCH = 2
CHANNELS = 256
NUM_ANCHORS = 3
LEVEL_HW = [128, 64, 32, 16, 8]
STRIDES = [4, 8, 16, 32, 64]
SIZES = [32.0, 64.0, 128.0, 256.0, 512.0]
ASPECT_RATIOS = np.array([0.5, 1.0, 2.0], dtype=np.float32)
PRE_NMS_TOP_N = 2000
POST_NMS_TOP_N = 1000
NMS_THRESH = 0.7
MIN_SIZE = 0.001

LEVEL_N = [hw * hw * NUM_ANCHORS for hw in LEVEL_HW]
TOTAL = sum(LEVEL_N)
PAD_TOTAL = 65536
ROWS, LANES = 512, 128
OUT_ROWS = 8
NEG_INF = float("-inf")


def _np_anchors():
    alla = []
    for hw, stride, size in zip(LEVEL_HW, STRIDES, SIZES):
        ar = ASPECT_RATIOS
        h_ratios = np.sqrt(ar)
        w_ratios = (np.float32(1.0) / h_ratios).astype(np.float32)
        ws = (w_ratios * np.float32(size)).astype(np.float32)
        hs = (h_ratios * np.float32(size)).astype(np.float32)
        base = np.round(np.stack([-ws, -hs, ws, hs], axis=1) / np.float32(2.0)).astype(np.float32)
        shifts = (np.arange(hw, dtype=np.float32) * np.float32(stride)).astype(np.float32)
        sy, sx = np.meshgrid(shifts, shifts, indexing="ij")
        shift = np.stack([sx.ravel(), sy.ravel(), sx.ravel(), sy.ravel()], axis=1).astype(np.float32)
        alla.append((shift[:, None, :] + base[None, :, :]).reshape(-1, 4))
    return np.concatenate(alla, axis=0).astype(np.float32)


_ANCHORS = _np_anchors()
_AW = (_ANCHORS[:, 2] - _ANCHORS[:, 0]).astype(np.float32)
_AH = (_ANCHORS[:, 3] - _ANCHORS[:, 1]).astype(np.float32)
_ACX = (_ANCHORS[:, 0] + np.float32(0.5) * _AW).astype(np.float32)
_ACY = (_ANCHORS[:, 1] + np.float32(0.5) * _AH).astype(np.float32)


def _pad_const(v, fill):
    out = np.full((PAD_TOTAL,), fill, dtype=np.float32)
    out[:TOTAL] = v
    return out


_AW_P = _pad_const(_AW, 1.0)
_AH_P = _pad_const(_AH, 1.0)
_ACX_P = _pad_const(_ACX, 0.0)
_ACY_P = _pad_const(_ACY, 0.0)

_LVL = np.concatenate([np.full((n,), i, dtype=np.float32) for i, n in enumerate(LEVEL_N)])
_LVL_P = _pad_const(_LVL, 0.0).reshape(ROWS, LANES)
_VALID_P = _pad_const(np.ones((TOTAL,), np.float32), 0.0).reshape(ROWS, LANES)

_L0_ROWS = LEVEL_N[0] // LANES
_L1_ROWS = LEVEL_N[1] // LANES
_L2_ROWS = LEVEL_N[2] // LANES
_TAIL_ROWS = ROWS - _L0_ROWS - _L1_ROWS - _L2_ROWS

_INT_MIN = np.int32(-(2**31))
_INT_MAX = np.int32(2**31 - 1)


def _conv2d(x, w, b):
    out = jax.lax.conv_general_dilated(
        x, w, (1, 1), "SAME", dimension_numbers=("NCHW", "OIHW", "NCHW"))
    return out + b[None, :, None, None]


def _permute_flatten(t, C):
    N, AxC, H, W = t.shape
    A = AxC // C
    t = t.reshape(N, A, C, H, W)
    t = jnp.transpose(t, (0, 3, 4, 1, 2))
    return t.reshape(N, -1, C)


def _decode_body(dx_ref, dy_ref, dw_ref, dh_ref, aw_ref, ah_ref, acx_ref, acy_ref,
                 x1_ref, y1_ref, x2_ref, y2_ref):
    aw, ah = aw_ref[...], ah_ref[...]
    acx, acy = acx_ref[...], acy_ref[...]
    clip = np.float32(np.log(1000.0 / 16.0))
    dw = jnp.minimum(dw_ref[...], clip)
    dh = jnp.minimum(dh_ref[...], clip)
    pcx = dx_ref[...] * aw + acx
    pcy = dy_ref[...] * ah + acy
    pw = jnp.exp(dw) * aw
    ph = jnp.exp(dh) * ah
    x1_ref[...] = pcx - 0.5 * pw
    y1_ref[...] = pcy - 0.5 * ph
    x2_ref[...] = pcx + 0.5 * pw
    y2_ref[...] = pcy + 0.5 * ph


def _decode(dx, dy, dw, dh):
    consts = [jnp.asarray(c).reshape(1, PAD_TOTAL)
              for c in (_AW_P, _AH_P, _ACX_P, _ACY_P)]
    full = pl.BlockSpec((BATCH, PAD_TOTAL), lambda: (0, 0))
    cblk = pl.BlockSpec((1, PAD_TOTAL), lambda: (0, 0))
    shp = jax.ShapeDtypeStruct((BATCH, PAD_TOTAL), jnp.float32)
    return pl.pallas_call(
        _decode_body,
        in_specs=[full, full, full, full, cblk, cblk, cblk, cblk],
        out_specs=[full, full, full, full],
        out_shape=[shp, shp, shp, shp],
    )(dx, dy, dw, dh, *consts)


def _sortable(x):
    b = jax.lax.bitcast_convert_type(x, jnp.int32)
    return jnp.where(b >= 0, b, jnp.bitwise_xor(jnp.invert(b), jnp.int32(_INT_MIN)))


def _kth_threshold(key, k):
    def body(_, lohi):
        lo, hi = lohi
        mid = (lo >> 1) + (hi >> 1) + (lo & hi & 1)
        c = jnp.sum((key > mid).astype(jnp.int32))
        big = c >= k
        return jnp.where(big, mid, lo), jnp.where(big, hi, mid)

    lo, hi = jax.lax.fori_loop(0, 33, body, (jnp.int32(_INT_MIN), jnp.int32(_INT_MAX)))
    return hi


def _level_topk_mask(seg, iota_seg, k):
    key = _sortable(seg)
    t = _kth_threshold(key, k)
    gt = key > t
    need = jnp.int32(k) - jnp.sum(gt.astype(jnp.int32))
    ties = key == t

    def body(_, lohi):
        lo, hi = lohi
        mid = (lo + hi) // 2
        c = jnp.sum((ties & (iota_seg < mid)).astype(jnp.int32))
        big = c >= need
        return jnp.where(big, lo, mid), jnp.where(big, mid, hi)

    lo, hi = jax.lax.fori_loop(0, 18, body, (jnp.int32(0), jnp.int32(seg.size)))
    sel = gt | (ties & (iota_seg < hi))
    return sel.astype(jnp.float32)


def _nms_body(obj_ref, s_ref, px1_ref, py1_ref, px2_ref, py2_ref, bw_ref, bh_ref,
              lvl_ref, vld_ref, ox1_ref, oy1_ref, ox2_ref, oy2_ref, osc_ref):
    obj = obj_ref[0]
    lin = jax.lax.broadcasted_iota(jnp.int32, (ROWS, LANES), 0) * LANES + \
        jax.lax.broadcasted_iota(jnp.int32, (ROWS, LANES), 1)

    r0 = 0
    masks = []
    for rows, k in ((_L0_ROWS, PRE_NMS_TOP_N), (_L1_ROWS, PRE_NMS_TOP_N),
                    (_L2_ROWS, PRE_NMS_TOP_N)):
        seg = obj[r0:r0 + rows]
        masks.append(_level_topk_mask(seg, lin[r0:r0 + rows] - r0 * LANES, k))
        r0 += rows
    masks.append(jnp.ones((_TAIL_ROWS, LANES), jnp.float32))
    topk = jnp.concatenate(masks, axis=0)

    bw, bh = bw_ref[...], bh_ref[...]
    x1 = jnp.clip(px1_ref[0], 0.0, bw)
    y1 = jnp.clip(py1_ref[0], 0.0, bh)
    x2 = jnp.clip(px2_ref[0], 0.0, bw)
    y2 = jnp.clip(py2_ref[0], 0.0, bh)
    ws, hs = x2 - x1, y2 - y1
    valid = (ws >= MIN_SIZE) & (hs >= MIN_SIZE) & (vld_ref[...] > 0.5) & (topk > 0.5)
    sc0 = jnp.where(valid, s_ref[0], NEG_INF)

    lvl = lvl_ref[...]
    offs = lvl * (jnp.maximum(bw, bh) + 1.0)
    x1o, y1o, x2o, y2o = x1 + offs, y1 + offs, x2 + offs, y2 + offs
    areas = ws * hs
    okey = _sortable(obj)
    oiota = jax.lax.broadcasted_iota(jnp.int32, (OUT_ROWS, LANES), 0) * LANES + \
        jax.lax.broadcasted_iota(jnp.int32, (OUT_ROWS, LANES), 1)
    zout = jnp.zeros((OUT_ROWS, LANES), jnp.float32)

    def body(i, state):
        sc, ox1, oy1, ox2, oy2, osc = state
        m = jnp.max(sc)
        alive = m > NEG_INF
        eq = sc >= m
        lmin = jnp.min(jnp.where(eq, lvl, 1e9))
        eq = eq & (lvl <= lmin)
        omax = jnp.max(jnp.where(eq, okey, jnp.int32(_INT_MIN)))
        eq = eq & (okey >= omax)
        best = jnp.min(jnp.where(eq, lin, jnp.int32(_INT_MAX)))
        onehot = lin == best
        pick = lambda a: jnp.sum(jnp.where(onehot, a, 0.0))
        bx1, by1, bx2, by2 = pick(x1), pick(y1), pick(x2), pick(y2)
        boff = pick(offs)
        barea = (bx2 - bx1) * (by2 - by1)
        xx1 = jnp.maximum(bx1 + boff, x1o)
        yy1 = jnp.maximum(by1 + boff, y1o)
        xx2 = jnp.minimum(bx2 + boff, x2o)
        yy2 = jnp.minimum(by2 + boff, y2o)
        inter = jnp.maximum(xx2 - xx1, 0.0) * jnp.maximum(yy2 - yy1, 0.0)
        iou = inter / (barea + areas - inter + 1e-9)
        sup = (iou > NMS_THRESH) | onehot
        sc = jnp.where(sup & alive, NEG_INF, sc)
        slot = oiota == i
        put = lambda acc, v: acc + jnp.where(slot, jnp.where(alive, v, 0.0), 0.0)
        return (sc, put(ox1, bx1), put(oy1, by1), put(ox2, bx2), put(oy2, by2),
                osc + jnp.where(slot & alive, m, 0.0))

    _, ox1, oy1, ox2, oy2, osc = jax.lax.fori_loop(
        0, POST_NMS_TOP_N, body, (sc0, zout, zout, zout, zout, zout))
    ox1_ref[0], oy1_ref[0], ox2_ref[0], oy2_ref[0], osc_ref[0] = ox1, oy1, ox2, oy2, osc


def _nms(obj_r, s_r, px1, py1, px2, py2, bw_arr, bh_arr):
    img = pl.BlockSpec((1, ROWS, LANES), lambda b: (b, 0, 0))
    cst = pl.BlockSpec((ROWS, LANES), lambda b: (0, 0))
    oblk = pl.BlockSpec((1, OUT_ROWS, LANES), lambda b: (b, 0, 0))
    oshp = jax.ShapeDtypeStruct((BATCH, OUT_ROWS, LANES), jnp.float32)
    return pl.pallas_call(
        _nms_body,
        grid=(BATCH,),
        in_specs=[img, img, img, img, img, img, cst, cst, cst, cst],
        out_specs=[oblk, oblk, oblk, oblk, oblk],
        out_shape=[oshp, oshp, oshp, oshp, oshp],
    )(obj_r, s_r, px1, py1, px2, py2, bw_arr, bh_arr,
      jnp.asarray(_LVL_P), jnp.asarray(_VALID_P))


def kernel(feat0, feat1, feat2, feat3, feat4, conv_w, conv_b, cls_w, cls_b,
           bbox_w, bbox_b, image_h, image_w):
    feats = [feat0, feat1, feat2, feat3, feat4]
    N = BATCH
    obj_l, delta_l = [], []
    for f in feats:
        t = jax.nn.relu(_conv2d(f, conv_w, conv_b))
        obj_l.append(_conv2d(t, cls_w, cls_b))
        delta_l.append(_conv2d(t, bbox_w, bbox_b))
    obj_full = jnp.concatenate(
        [_permute_flatten(o, 1) for o in obj_l], axis=1).reshape(N, TOTAL)
    deltas_full = jnp.concatenate(
        [_permute_flatten(d, 4) for d in delta_l], axis=1)
    scores_full = jax.nn.sigmoid(obj_full)

    pad1 = ((0, 0), (0, PAD_TOTAL - TOTAL))
    dxyz = [jnp.pad(deltas_full[:, :, j], pad1) for j in range(4)]
    px1, py1, px2, py2 = _decode(*dxyz)
    proposals = jnp.stack([px1[:, :TOTAL], py1[:, :TOTAL],
                           px2[:, :TOTAL], py2[:, :TOTAL]], axis=-1)

    img_w_f = jnp.asarray(image_w, dtype=jnp.float32)
    img_h_f = jnp.asarray(image_h, dtype=jnp.float32)
    bw_arr = jnp.full((ROWS, LANES), img_w_f)
    bh_arr = jnp.full((ROWS, LANES), img_h_f)
    obj_pad = jnp.pad(obj_full, pad1, constant_values=0.0)
    s_pad = jnp.pad(scores_full, pad1, constant_values=0.0)
    rs = lambda a: a.reshape(N, ROWS, LANES)
    ox1, oy1, ox2, oy2, osc = _nms(rs(obj_pad), rs(s_pad), rs(px1), rs(py1),
                                   rs(px2), rs(py2), bw_arr, bh_arr)
    fb = lambda a: a.reshape(N, OUT_ROWS * LANES)[:, :POST_NMS_TOP_N]
    final_boxes = jnp.stack([fb(ox1), fb(oy1), fb(ox2), fb(oy2)], axis=-1)
    final_scores = fb(osc)

    anchors_exp = jnp.broadcast_to(jnp.asarray(_ANCHORS)[None], (N, TOTAL, 4))
    return (final_boxes, final_scores, proposals, anchors_exp,
            obj_full.reshape(N, TOTAL, 1), deltas_full)
